# Initial kernel scaffold; baseline (speedup 1.0000x reference)
import jax
import jax.numpy as jnp
from jax.experimental import pallas as pl


def kernel(u, i, graph_x, graph_edge_index, user_table, item_table, W1, b1, W2, b2):
    raise NotImplementedError("write your pallas kernel here")


# speedup vs baseline: 10.9884x; 10.9884x over previous
import functools

import jax
import jax.numpy as jnp
from jax import lax
from jax.experimental import pallas as pl
from jax.experimental.pallas import tpu as pltpu
from jax.experimental.pallas import tpu_sc as plsc

_EPS = 1e-15
_MAXNORM = 1.0 - 1e-5

_NC = 2
_NS = 16
_NW = _NC * _NS
_CW = 125

_N_ITEMS = 10000
_N_PAD = 10240
_N_EDGES = 640000
_B = 1024
_CHUNKS = _N_EDGES // (_NW * _CW)
_UB = _B // _NW
_RSTR = _N_PAD // _NS


def _artanh(x):
    x = jnp.clip(x, -1.0 + 1e-7, 1.0 - 1e-7)
    return 0.5 * jnp.log((1.0 + x) / (1.0 - x))


def _norm(x):
    return jnp.maximum(jnp.sqrt(jnp.sum(x * x, axis=-1, keepdims=True)), _EPS)


def _proj(x):
    n = _norm(x)
    return jnp.where(n > _MAXNORM, x / n * _MAXNORM, x)


def _expmap0(u):
    n = _norm(u)
    return _proj(jnp.tanh(n) * u / n)


def _logmap0(p):
    n = _norm(p)
    return _artanh(n) * p / n


def _mobius_add(x, y):
    x2 = jnp.sum(x * x, axis=-1, keepdims=True)
    y2 = jnp.sum(y * y, axis=-1, keepdims=True)
    xy = jnp.sum(x * y, axis=-1, keepdims=True)
    num = (1.0 + 2.0 * xy + y2) * x + (1.0 - x2) * y
    den = 1.0 + 2.0 * xy + x2 * y2
    return num / jnp.maximum(den, _EPS)


def _matmul_nt(a, b):
    return lax.dot_general(a, b, (((1,), (1,)), ((), ())),
                           preferred_element_type=jnp.float32,
                           precision=lax.Precision.HIGHEST)


def _matmul_nn(a, b):
    return lax.dot_general(a, b, (((1,), (0,)), ((), ())),
                           preferred_element_type=jnp.float32,
                           precision=lax.Precision.HIGHEST)


def _pre_agg(x_hyp, w, b):
    xn = _norm(x_hyp)
    mx = _matmul_nt(x_hyp, w)
    mxn = _norm(mx)
    mv = _proj(jnp.tanh(mxn / xn * _artanh(xn)) * mx / mxn)
    bias_h = _proj(_expmap0(b))
    h = _proj(_mobius_add(mv, bias_h))
    return _logmap0(h)


def _post_agg(acc, deg):
    deg = deg[0, :, :1] + deg[1, :, :1]
    agg = (acc[0] + acc[1]) / jnp.maximum(deg, 1.0)
    h2 = _proj(_expmap0(agg))
    return _proj(_expmap0(jnp.maximum(_logmap0(h2), 0.0)))


def _tc_pre1_body(x_ref, w_ref, b_ref, o_ref):
    x_hyp = _proj(_expmap0(x_ref[...]))
    o_ref[...] = _pre_agg(x_hyp, w_ref[...], b_ref[...])


def _tc_mid_body(acc_ref, deg_ref, w_ref, b_ref, o_ref):
    x_hyp = _post_agg(acc_ref[...], deg_ref[...])
    o_ref[...] = _pre_agg(x_hyp, w_ref[...], b_ref[...])


def _tc_final_body(acc_ref, deg_ref, i_ref, u_ref, o_ref, h_v):
    k = pl.program_id(0)

    @pl.when(k == 0)
    def _():
        h_v[...] = _post_agg(acc_ref[:, :_N_ITEMS], deg_ref[:, :_N_ITEMS])

    i_emb = _matmul_nn(i_ref[...], h_v[...])
    pu = _proj(_expmap0(u_ref[...]))
    pi = _proj(i_emb)
    ma = _mobius_add(-pu, pi)
    n = jnp.sqrt(jnp.maximum(jnp.sum(ma * ma, axis=-1, keepdims=True),
                             _EPS * _EPS))
    d = 2.0 * _artanh(n)
    d2 = d * d
    o_ref[...] = 1.0 / (jnp.exp(d2 - 2.0) + 1.0)


def _run_pre1(item_table, w1, b1):
    rb = 1000
    grid = _N_ITEMS // rb
    return pl.pallas_call(
        _tc_pre1_body,
        grid=(grid,),
        in_specs=[
            pl.BlockSpec((rb, item_table.shape[1]), lambda k: (k, 0)),
            pl.BlockSpec(w1.shape, lambda k: (0, 0)),
            pl.BlockSpec(b1.shape, lambda k: (0, 0)),
        ],
        out_specs=pl.BlockSpec((rb, w1.shape[0]), lambda k: (k, 0)),
        out_shape=jax.ShapeDtypeStruct((_N_ITEMS, w1.shape[0]), jnp.float32),
    )(item_table, w1, b1)


def _run_mid(acc, deg, w2, b2):
    rb = 1000
    grid = _N_ITEMS // rb
    return pl.pallas_call(
        _tc_mid_body,
        grid=(grid,),
        in_specs=[
            pl.BlockSpec((_NC, rb, 64), lambda k: (0, k, 0)),
            pl.BlockSpec((_NC, rb, 16), lambda k: (0, k, 0)),
            pl.BlockSpec(w2.shape, lambda k: (0, 0)),
            pl.BlockSpec(b2.shape, lambda k: (0, 0)),
        ],
        out_specs=pl.BlockSpec((rb, w2.shape[0]), lambda k: (k, 0)),
        out_shape=jax.ShapeDtypeStruct((_N_ITEMS, w2.shape[0]), jnp.float32),
    )(acc, deg, w2, b2)


def _run_final(acc, deg, i, u_emb):
    bb = 128
    grid = _B // bb
    out = pl.pallas_call(
        _tc_final_body,
        grid=(grid,),
        in_specs=[
            pl.BlockSpec((_NC, _N_PAD, 64), lambda k: (0, 0, 0)),
            pl.BlockSpec((_NC, _N_PAD, 16), lambda k: (0, 0, 0)),
            pl.BlockSpec((bb, _N_ITEMS), lambda k: (k, 0)),
            pl.BlockSpec((bb, 64), lambda k: (k, 0)),
        ],
        out_specs=pl.BlockSpec((bb, 1), lambda k: (k, 0)),
        out_shape=jax.ShapeDtypeStruct((_B, 1), jnp.float32),
        scratch_shapes=[pltpu.VMEM((_N_ITEMS, 64), jnp.float32)],
    )(acc, deg, i, u_emb)
    return out[:, 0]


def _sc1_body(xt_hbm, src_hbm, dst_hbm, uidx_hbm, utab_hbm, z64_hbm, z16_hbm,
              ones_hbm, acc_hbm, deg_hbm, uemb_hbm,
              src_v, dst_v, rows_v, ones_v, uidx_v, urows_v, acc_sh, deg_sh, sem):
    cid = lax.axis_index("c")
    sid = lax.axis_index("s")
    wid = sid * _NC + cid
    r0 = sid * _RSTR
    pltpu.sync_copy(z64_hbm.at[pl.ds(r0, _RSTR)], acc_sh.at[pl.ds(r0, _RSTR)])
    pltpu.sync_copy(z16_hbm.at[pl.ds(r0, _RSTR)], deg_sh.at[pl.ds(r0, _RSTR)])
    pltpu.sync_copy(src_hbm.at[wid], src_v)
    pltpu.sync_copy(dst_hbm.at[wid], dst_v)
    pltpu.sync_copy(ones_hbm, ones_v)
    plsc.subcore_barrier()

    def chunk(j, carry):
        pltpu.async_copy(xt_hbm.at[src_v.at[j]], rows_v, sem).wait()
        pltpu.sync_copy(rows_v, acc_sh.at[dst_v.at[j]], add=True)
        pltpu.sync_copy(ones_v, deg_sh.at[dst_v.at[j]], add=True)
        return carry

    lax.fori_loop(0, _CHUNKS, chunk, 0)

    pltpu.sync_copy(uidx_hbm.at[wid], uidx_v)
    pltpu.async_copy(utab_hbm.at[uidx_v], urows_v, sem).wait()
    pltpu.sync_copy(urows_v, uemb_hbm.at[pl.ds(wid * _UB, _UB)])

    plsc.subcore_barrier()
    pltpu.sync_copy(acc_sh.at[pl.ds(r0, _RSTR)], acc_hbm.at[cid, pl.ds(r0, _RSTR)])
    pltpu.sync_copy(deg_sh.at[pl.ds(r0, _RSTR)], deg_hbm.at[cid, pl.ds(r0, _RSTR)])


def _sc2_body(xt_hbm, src_hbm, dst_hbm, z64_hbm, acc_hbm,
              src_v, dst_v, rows_v, acc_sh, sem):
    cid = lax.axis_index("c")
    sid = lax.axis_index("s")
    wid = sid * _NC + cid
    r0 = sid * _RSTR
    pltpu.sync_copy(z64_hbm.at[pl.ds(r0, _RSTR)], acc_sh.at[pl.ds(r0, _RSTR)])
    pltpu.sync_copy(src_hbm.at[wid], src_v)
    pltpu.sync_copy(dst_hbm.at[wid], dst_v)
    plsc.subcore_barrier()

    def chunk(j, carry):
        pltpu.async_copy(xt_hbm.at[src_v.at[j]], rows_v, sem).wait()
        pltpu.sync_copy(rows_v, acc_sh.at[dst_v.at[j]], add=True)
        return carry

    lax.fori_loop(0, _CHUNKS, chunk, 0)

    plsc.subcore_barrier()
    pltpu.sync_copy(acc_sh.at[pl.ds(r0, _RSTR)], acc_hbm.at[cid, pl.ds(r0, _RSTR)])


def _sc_mesh():
    return plsc.VectorSubcoreMesh(core_axis_name="c", subcore_axis_name="s",
                                  num_cores=_NC, num_subcores=_NS)


def _run_sc1(xt, src3, dst3, uidx, user_table, z64, z16, ones):
    f32 = jnp.float32
    call = pl.kernel(
        _sc1_body,
        out_type=[
            jax.ShapeDtypeStruct((_NC, _N_PAD, 64), f32),
            jax.ShapeDtypeStruct((_NC, _N_PAD, 16), f32),
            jax.ShapeDtypeStruct((_B, 64), f32),
        ],
        mesh=_sc_mesh(),
        compiler_params=pltpu.CompilerParams(use_tc_tiling_on_sc=False),
        scratch_types=[
            pltpu.VMEM((_CHUNKS, _CW), jnp.int32),
            pltpu.VMEM((_CHUNKS, _CW), jnp.int32),
            pltpu.VMEM((_CW, 64), f32),
            pltpu.VMEM((_CW, 16), f32),
            pltpu.VMEM((_UB,), jnp.int32),
            pltpu.VMEM((_UB, 64), f32),
            pltpu.VMEM_SHARED((_N_PAD, 64), f32),
            pltpu.VMEM_SHARED((_N_PAD, 16), f32),
            pltpu.SemaphoreType.DMA,
        ],
    )
    return call(xt, src3, dst3, uidx, user_table, z64, z16, ones)


def _run_sc2(xt, src3, dst3, z64):
    f32 = jnp.float32
    call = pl.kernel(
        _sc2_body,
        out_type=jax.ShapeDtypeStruct((_NC, _N_PAD, 64), f32),
        mesh=_sc_mesh(),
        compiler_params=pltpu.CompilerParams(use_tc_tiling_on_sc=False),
        scratch_types=[
            pltpu.VMEM((_CHUNKS, _CW), jnp.int32),
            pltpu.VMEM((_CHUNKS, _CW), jnp.int32),
            pltpu.VMEM((_CW, 64), f32),
            pltpu.VMEM_SHARED((_N_PAD, 64), f32),
            pltpu.SemaphoreType.DMA,
        ],
    )
    return call(xt, src3, dst3, z64)


def kernel(u, i, graph_x, graph_edge_index, user_table, item_table, W1, b1, W2, b2):
    del graph_x
    f32 = jnp.float32
    src3 = graph_edge_index[0].astype(jnp.int32).reshape(_NW, _CHUNKS, _CW)
    dst3 = graph_edge_index[1].astype(jnp.int32).reshape(_NW, _CHUNKS, _CW)
    uidx = u.astype(jnp.int32).reshape(_NW, _UB)
    z64 = jnp.zeros((_N_PAD, 64), f32)
    z16 = jnp.zeros((_N_PAD, 16), f32)
    ones = jnp.ones((_CW, 16), f32)
    b1r = b1.reshape(1, -1).astype(f32)
    b2r = b2.reshape(1, -1).astype(f32)

    xt1 = _run_pre1(item_table.astype(f32), W1.astype(f32), b1r)
    acc1, deg, u_emb = _run_sc1(xt1, src3, dst3, uidx, user_table.astype(f32),
                                z64, z16, ones)
    xt2 = _run_mid(acc1, deg, W2.astype(f32), b2r)
    acc2 = _run_sc2(xt2, src3, dst3, z64)
    return _run_final(acc2, deg, i.astype(f32), u_emb)


# speedup vs baseline: 15.0216x; 1.3670x over previous
import functools

import jax
import jax.numpy as jnp
from jax import lax
from jax.experimental import pallas as pl
from jax.experimental.pallas import tpu as pltpu
from jax.experimental.pallas import tpu_sc as plsc

_EPS = 1e-15
_MAXNORM = 1.0 - 1e-5

_NC = 2
_NS = 16
_NW = _NC * _NS
_CW = 125

_N_ITEMS = 10000
_N_PAD = 10240
_N_EDGES = 640000
_B = 1024
_CHUNKS = _N_EDGES // (_NW * _CW)
_UB = _B // _NW
_RSTR = _N_PAD // _NS


def _artanh(x):
    x = jnp.clip(x, -1.0 + 1e-7, 1.0 - 1e-7)
    return 0.5 * jnp.log((1.0 + x) / (1.0 - x))


def _norm(x):
    return jnp.maximum(jnp.sqrt(jnp.sum(x * x, axis=-1, keepdims=True)), _EPS)


def _proj(x):
    n = _norm(x)
    return jnp.where(n > _MAXNORM, x / n * _MAXNORM, x)


def _expmap0(u):
    n = _norm(u)
    return _proj(jnp.tanh(n) * u / n)


def _logmap0(p):
    n = _norm(p)
    return _artanh(n) * p / n


def _mobius_add(x, y):
    x2 = jnp.sum(x * x, axis=-1, keepdims=True)
    y2 = jnp.sum(y * y, axis=-1, keepdims=True)
    xy = jnp.sum(x * y, axis=-1, keepdims=True)
    num = (1.0 + 2.0 * xy + y2) * x + (1.0 - x2) * y
    den = 1.0 + 2.0 * xy + x2 * y2
    return num / jnp.maximum(den, _EPS)


def _matmul_nt(a, b):
    return lax.dot_general(a, b, (((1,), (1,)), ((), ())),
                           preferred_element_type=jnp.float32,
                           precision=lax.Precision.HIGHEST)


def _matmul_nn(a, b):
    return lax.dot_general(a, b, (((1,), (0,)), ((), ())),
                           preferred_element_type=jnp.float32,
                           precision=lax.Precision.HIGHEST)


def _pre_agg(x_hyp, w, b):
    xn = _norm(x_hyp)
    mx = _matmul_nt(x_hyp, w)
    mxn = _norm(mx)
    mv = _proj(jnp.tanh(mxn / xn * _artanh(xn)) * mx / mxn)
    bias_h = _proj(_expmap0(b))
    h = _proj(_mobius_add(mv, bias_h))
    return _logmap0(h)


def _post_agg(acc, deg):
    deg = deg[0, :, :1] + deg[1, :, :1]
    agg = (acc[0] + acc[1]) / jnp.maximum(deg, 1.0)
    h2 = _proj(_expmap0(agg))
    return _proj(_expmap0(jnp.maximum(_logmap0(h2), 0.0)))


def _tc_pre1_body(x_ref, w_ref, b_ref, o_ref):
    x_hyp = _proj(_expmap0(x_ref[...]))
    o_ref[...] = _pre_agg(x_hyp, w_ref[...], b_ref[...])


def _tc_mid_body(acc_ref, deg_ref, w_ref, b_ref, o_ref):
    x_hyp = _post_agg(acc_ref[...], deg_ref[...])
    o_ref[...] = _pre_agg(x_hyp, w_ref[...], b_ref[...])


def _tc_final_body(acc_ref, deg_ref, i_ref, u_ref, o_ref, h_v):
    k = pl.program_id(0)

    @pl.when(k == 0)
    def _():
        h_v[...] = _post_agg(acc_ref[:, :_N_ITEMS], deg_ref[:, :_N_ITEMS])

    i_emb = _matmul_nn(i_ref[...], h_v[...])
    pu = _proj(_expmap0(u_ref[...]))
    pi = _proj(i_emb)
    ma = _mobius_add(-pu, pi)
    n = jnp.sqrt(jnp.maximum(jnp.sum(ma * ma, axis=-1, keepdims=True),
                             _EPS * _EPS))
    d = 2.0 * _artanh(n)
    d2 = d * d
    o_ref[...] = 1.0 / (jnp.exp(d2 - 2.0) + 1.0)


def _run_pre1(item_table, w1, b1):
    rb = 1000
    grid = _N_ITEMS // rb
    return pl.pallas_call(
        _tc_pre1_body,
        grid=(grid,),
        in_specs=[
            pl.BlockSpec((rb, item_table.shape[1]), lambda k: (k, 0)),
            pl.BlockSpec(w1.shape, lambda k: (0, 0)),
            pl.BlockSpec(b1.shape, lambda k: (0, 0)),
        ],
        out_specs=pl.BlockSpec((rb, w1.shape[0]), lambda k: (k, 0)),
        out_shape=jax.ShapeDtypeStruct((_N_ITEMS, w1.shape[0]), jnp.float32),
    )(item_table, w1, b1)


def _run_mid(acc, deg, w2, b2):
    rb = 1000
    grid = _N_ITEMS // rb
    return pl.pallas_call(
        _tc_mid_body,
        grid=(grid,),
        in_specs=[
            pl.BlockSpec((_NC, rb, 64), lambda k: (0, k, 0)),
            pl.BlockSpec((_NC, rb, 16), lambda k: (0, k, 0)),
            pl.BlockSpec(w2.shape, lambda k: (0, 0)),
            pl.BlockSpec(b2.shape, lambda k: (0, 0)),
        ],
        out_specs=pl.BlockSpec((rb, w2.shape[0]), lambda k: (k, 0)),
        out_shape=jax.ShapeDtypeStruct((_N_ITEMS, w2.shape[0]), jnp.float32),
    )(acc, deg, w2, b2)


def _run_final(acc, deg, i, u_emb):
    bb = 128
    grid = _B // bb
    out = pl.pallas_call(
        _tc_final_body,
        grid=(grid,),
        in_specs=[
            pl.BlockSpec((_NC, _N_PAD, 64), lambda k: (0, 0, 0)),
            pl.BlockSpec((_NC, _N_PAD, 16), lambda k: (0, 0, 0)),
            pl.BlockSpec((bb, _N_ITEMS), lambda k: (k, 0)),
            pl.BlockSpec((bb, 64), lambda k: (k, 0)),
        ],
        out_specs=pl.BlockSpec((bb, 1), lambda k: (k, 0)),
        out_shape=jax.ShapeDtypeStruct((_B, 1), jnp.float32),
        scratch_shapes=[pltpu.VMEM((_N_ITEMS, 64), jnp.float32)],
    )(acc, deg, i, u_emb)
    return out[:, 0]


def _agg_loop(xt_hbm, src_v, dst_v, rows0, rows1, acc_sh, sem_g, sem_s,
              ones_v=None, deg_sh=None):
    nb2 = _CHUNKS // 2

    def gather(j, buf):
        return pltpu.async_copy(xt_hbm.at[src_v.at[j]], buf, sem_g)

    gather(0, rows0)
    gather(1, rows1)

    def step(t, carry):
        j = 2 * t
        pltpu.make_async_copy(xt_hbm.at[src_v.at[j]], rows0, sem_g).wait()
        sa = pltpu.async_copy(rows0, acc_sh.at[dst_v.at[j]], sem_s, add=True)
        if deg_sh is not None:
            sd = pltpu.async_copy(ones_v, deg_sh.at[dst_v.at[j]], sem_s, add=True)
        pltpu.make_async_copy(xt_hbm.at[src_v.at[j]], rows1, sem_g).wait()
        sa.wait()
        if deg_sh is not None:
            sd.wait()

        @pl.when(t < nb2 - 1)
        def _():
            gather(j + 2, rows0)

        sa1 = pltpu.async_copy(rows1, acc_sh.at[dst_v.at[j + 1]], sem_s, add=True)
        if deg_sh is not None:
            sd1 = pltpu.async_copy(ones_v, deg_sh.at[dst_v.at[j + 1]], sem_s,
                                   add=True)
        sa1.wait()
        if deg_sh is not None:
            sd1.wait()

        @pl.when(t < nb2 - 1)
        def _():
            gather(j + 3, rows1)

        return carry

    lax.fori_loop(0, nb2, step, 0)


def _sc1_body(xt_hbm, src_hbm, dst_hbm, uidx_hbm, utab_hbm, z64_hbm, z16_hbm,
              ones_hbm, acc_hbm, deg_hbm, uemb_hbm,
              src_v, dst_v, rows0, rows1, ones_v, uidx_v, urows_v,
              acc_sh, deg_sh, sem_g, sem_s):
    cid = lax.axis_index("c")
    sid = lax.axis_index("s")
    wid = sid * _NC + cid
    r0 = sid * _RSTR
    pltpu.sync_copy(z64_hbm.at[pl.ds(r0, _RSTR)], acc_sh.at[pl.ds(r0, _RSTR)])
    pltpu.sync_copy(z16_hbm.at[pl.ds(r0, _RSTR)], deg_sh.at[pl.ds(r0, _RSTR)])
    pltpu.sync_copy(src_hbm.at[wid], src_v)
    pltpu.sync_copy(dst_hbm.at[wid], dst_v)
    pltpu.sync_copy(ones_hbm, ones_v)
    pltpu.sync_copy(uidx_hbm.at[wid], uidx_v)
    pltpu.async_copy(utab_hbm.at[uidx_v], urows_v, sem_g).wait()
    pltpu.sync_copy(urows_v, uemb_hbm.at[pl.ds(wid * _UB, _UB)])
    plsc.subcore_barrier()

    _agg_loop(xt_hbm, src_v, dst_v, rows0, rows1, acc_sh, sem_g, sem_s,
              ones_v=ones_v, deg_sh=deg_sh)

    plsc.subcore_barrier()
    pltpu.sync_copy(acc_sh.at[pl.ds(r0, _RSTR)], acc_hbm.at[cid, pl.ds(r0, _RSTR)])
    pltpu.sync_copy(deg_sh.at[pl.ds(r0, _RSTR)], deg_hbm.at[cid, pl.ds(r0, _RSTR)])


def _sc2_body(xt_hbm, src_hbm, dst_hbm, z64_hbm, acc_hbm,
              src_v, dst_v, rows0, rows1, acc_sh, sem_g, sem_s):
    cid = lax.axis_index("c")
    sid = lax.axis_index("s")
    wid = sid * _NC + cid
    r0 = sid * _RSTR
    pltpu.sync_copy(z64_hbm.at[pl.ds(r0, _RSTR)], acc_sh.at[pl.ds(r0, _RSTR)])
    pltpu.sync_copy(src_hbm.at[wid], src_v)
    pltpu.sync_copy(dst_hbm.at[wid], dst_v)
    plsc.subcore_barrier()

    _agg_loop(xt_hbm, src_v, dst_v, rows0, rows1, acc_sh, sem_g, sem_s)

    plsc.subcore_barrier()
    pltpu.sync_copy(acc_sh.at[pl.ds(r0, _RSTR)], acc_hbm.at[cid, pl.ds(r0, _RSTR)])


def _sc_mesh():
    return plsc.VectorSubcoreMesh(core_axis_name="c", subcore_axis_name="s",
                                  num_cores=_NC, num_subcores=_NS)


def _run_sc1(xt, src3, dst3, uidx, user_table, z64, z16, ones):
    f32 = jnp.float32
    call = pl.kernel(
        _sc1_body,
        out_type=[
            jax.ShapeDtypeStruct((_NC, _N_PAD, 64), f32),
            jax.ShapeDtypeStruct((_NC, _N_PAD, 16), f32),
            jax.ShapeDtypeStruct((_B, 64), f32),
        ],
        mesh=_sc_mesh(),
        compiler_params=pltpu.CompilerParams(use_tc_tiling_on_sc=False),
        scratch_types=[
            pltpu.VMEM((_CHUNKS, _CW), jnp.int32),
            pltpu.VMEM((_CHUNKS, _CW), jnp.int32),
            pltpu.VMEM((_CW, 64), f32),
            pltpu.VMEM((_CW, 64), f32),
            pltpu.VMEM((_CW, 16), f32),
            pltpu.VMEM((_UB,), jnp.int32),
            pltpu.VMEM((_UB, 64), f32),
            pltpu.VMEM_SHARED((_N_PAD, 64), f32),
            pltpu.VMEM_SHARED((_N_PAD, 16), f32),
            pltpu.SemaphoreType.DMA,
            pltpu.SemaphoreType.DMA,
        ],
    )
    return call(xt, src3, dst3, uidx, user_table, z64, z16, ones)


def _run_sc2(xt, src3, dst3, z64):
    f32 = jnp.float32
    call = pl.kernel(
        _sc2_body,
        out_type=jax.ShapeDtypeStruct((_NC, _N_PAD, 64), f32),
        mesh=_sc_mesh(),
        compiler_params=pltpu.CompilerParams(use_tc_tiling_on_sc=False),
        scratch_types=[
            pltpu.VMEM((_CHUNKS, _CW), jnp.int32),
            pltpu.VMEM((_CHUNKS, _CW), jnp.int32),
            pltpu.VMEM((_CW, 64), f32),
            pltpu.VMEM((_CW, 64), f32),
            pltpu.VMEM_SHARED((_N_PAD, 64), f32),
            pltpu.SemaphoreType.DMA,
            pltpu.SemaphoreType.DMA,
        ],
    )
    return call(xt, src3, dst3, z64)


def kernel(u, i, graph_x, graph_edge_index, user_table, item_table, W1, b1, W2, b2):
    del graph_x
    f32 = jnp.float32
    src3 = graph_edge_index[0].astype(jnp.int32).reshape(_NW, _CHUNKS, _CW)
    dst3 = graph_edge_index[1].astype(jnp.int32).reshape(_NW, _CHUNKS, _CW)
    uidx = u.astype(jnp.int32).reshape(_NW, _UB)
    z64 = jnp.zeros((_N_PAD, 64), f32)
    z16 = jnp.zeros((_N_PAD, 16), f32)
    ones = jnp.ones((_CW, 16), f32)
    b1r = b1.reshape(1, -1).astype(f32)
    b2r = b2.reshape(1, -1).astype(f32)

    xt1 = _run_pre1(item_table.astype(f32), W1.astype(f32), b1r)
    acc1, deg, u_emb = _run_sc1(xt1, src3, dst3, uidx, user_table.astype(f32),
                                z64, z16, ones)
    xt2 = _run_mid(acc1, deg, W2.astype(f32), b2r)
    acc2 = _run_sc2(xt2, src3, dst3, z64)
    return _run_final(acc2, deg, i.astype(f32), u_emb)
